# Initial kernel scaffold; baseline (speedup 1.0000x reference)
import functools

import jax
import jax.numpy as jnp
from jax import lax
from jax.experimental import pallas as pl
from jax.experimental.pallas import tpu as pltpu
from jax.experimental.pallas import tpu_sc as plsc

N = 10000
NPAD = 10240
E = 320000
EPAD = 327680
DIN = 128
DH = 64
NC, NS, L = 2, 16, 16
CH = 128
NCHUNK = EPAD // CH
DEN_CH = NCHUNK // NS
MSG_CH = NCHUNK // (NC * NS)
SLICE = NPAD // NS
RB = 8
BLK = NPAD // RB

_f32 = jnp.float32
_i32 = jnp.int32


def _dense_in_body(x_ref, w_ref, a2_ref, h_ref, asad_ref):
    h = jnp.dot(x_ref[...], w_ref[...], preferred_element_type=_f32)
    h_ref[...] = h
    asad_ref[...] = jnp.dot(h, a2_ref[...], preferred_element_type=_f32)


def _dense_in(x, wT, a2):
    return pl.pallas_call(
        _dense_in_body,
        grid=(RB,),
        in_specs=[
            pl.BlockSpec((BLK, DIN), lambda i: (i, 0)),
            pl.BlockSpec((DIN, DH), lambda i: (0, 0)),
            pl.BlockSpec((DH, 2), lambda i: (0, 0)),
        ],
        out_specs=[
            pl.BlockSpec((BLK, DH), lambda i: (i, 0)),
            pl.BlockSpec((BLK, 2), lambda i: (i, 0)),
        ],
        out_shape=[
            jax.ShapeDtypeStruct((NPAD, DH), _f32),
            jax.ShapeDtypeStruct((NPAD, 2), _f32),
        ],
    )(x, wT, a2)


def _post_ln(acc_ref, b_ref, g_ref, be_ref):
    r = acc_ref[0] + acc_ref[1] + b_ref[...]
    r = jnp.maximum(r, 0.0)
    mu = jnp.mean(r, axis=-1, keepdims=True)
    var = jnp.mean((r - mu) ** 2, axis=-1, keepdims=True)
    return (r - mu) / jnp.sqrt(var + 1e-5) * g_ref[...] + be_ref[...]


def _mid_body(acc_ref, b_ref, g_ref, be_ref, w_ref, a2_ref, h_ref, asad_ref):
    ln = _post_ln(acc_ref, b_ref, g_ref, be_ref)
    h = jnp.dot(ln, w_ref[...], preferred_element_type=_f32)
    h_ref[...] = h
    asad_ref[...] = jnp.dot(h, a2_ref[...], preferred_element_type=_f32)


def _mid(acc, b, g, be, wT, a2):
    return pl.pallas_call(
        _mid_body,
        grid=(RB,),
        in_specs=[
            pl.BlockSpec((NC, BLK, DH), lambda i: (0, i, 0)),
            pl.BlockSpec((1, DH), lambda i: (0, 0)),
            pl.BlockSpec((1, DH), lambda i: (0, 0)),
            pl.BlockSpec((1, DH), lambda i: (0, 0)),
            pl.BlockSpec((DH, DH), lambda i: (0, 0)),
            pl.BlockSpec((DH, 2), lambda i: (0, 0)),
        ],
        out_specs=[
            pl.BlockSpec((BLK, DH), lambda i: (i, 0)),
            pl.BlockSpec((BLK, 2), lambda i: (i, 0)),
        ],
        out_shape=[
            jax.ShapeDtypeStruct((NPAD, DH), _f32),
            jax.ShapeDtypeStruct((NPAD, 2), _f32),
        ],
    )(acc, b, g, be, wT, a2)


def _fin_body(acc_ref, b_ref, g_ref, be_ref, y_ref):
    y_ref[...] = _post_ln(acc_ref, b_ref, g_ref, be_ref)


def _fin(acc, b, g, be):
    return pl.pallas_call(
        _fin_body,
        grid=(RB,),
        in_specs=[
            pl.BlockSpec((NC, BLK, DH), lambda i: (0, i, 0)),
            pl.BlockSpec((1, DH), lambda i: (0, 0)),
            pl.BlockSpec((1, DH), lambda i: (0, 0)),
            pl.BlockSpec((1, DH), lambda i: (0, 0)),
        ],
        out_specs=pl.BlockSpec((BLK, DH), lambda i: (i, 0)),
        out_shape=jax.ShapeDtypeStruct((NPAD, DH), _f32),
    )(acc, b, g, be)


_mesh = plsc.VectorSubcoreMesh(core_axis_name="c", subcore_axis_name="s")


@functools.partial(
    pl.kernel,
    mesh=_mesh,
    out_type=jax.ShapeDtypeStruct((NC, NPAD, DH), _f32),
    scratch_types=[
        pltpu.VMEM((NPAD,), _f32),
        pltpu.VMEM((NPAD,), _f32),
        pltpu.VMEM((NPAD,), _f32),
        pltpu.VMEM((SLICE, DH), _f32),
        pltpu.VMEM((SLICE,), _f32),
        pltpu.VMEM((SLICE,), _f32),
        pltpu.VMEM((SLICE,), _f32),
        pltpu.VMEM((CH,), _i32),
        pltpu.VMEM((CH,), _i32),
        pltpu.VMEM((CH,), _f32),
        pltpu.VMEM((CH,), _f32),
        pltpu.VMEM((CH, DH), _f32),
        pltpu.VMEM_SHARED((NPAD,), _f32),
        pltpu.VMEM_SHARED((NPAD,), _f32),
        pltpu.VMEM_SHARED((NPAD, DH), _f32),
        pltpu.SemaphoreType.DMA,
    ],
)
def _gat_sc(src_hbm, dst_hbm, as_hbm, ad_hbm, h_hbm, acc_out,
            as_tab, ad_tab, inv_tab, hbuf, dbuf, lbuf, zbuf,
            srcc, dstc, exb, alpha, rows,
            denom_s, inv_s, acc_s, sem):
    cid = lax.axis_index("c")
    sid = lax.axis_index("s")
    wid = cid * NS + sid
    nbase = sid * SLICE

    pltpu.sync_copy(as_hbm, as_tab)
    pltpu.sync_copy(ad_hbm, ad_tab)

    def _mstep(i, carry):
        ma, mb = carry
        ma = jnp.maximum(ma, as_tab[pl.ds(i * L, L)])
        mb = jnp.maximum(mb, ad_tab[pl.ds(i * L, L)])
        return ma, mb
    ninf = jnp.full((L,), -3.0e38, _f32)
    ma, mb = lax.fori_loop(0, NPAD // L, _mstep, (ninf, ninf))
    m_shift = jnp.maximum(jnp.max(ma) + jnp.max(mb), 0.0)

    def _edge_ex(g):
        sidx = srcc[pl.ds(g * L, L)]
        didx = dstc[pl.ds(g * L, L)]
        s = plsc.load_gather(as_tab, [sidx]) + plsc.load_gather(ad_tab, [didx])
        return didx, jnp.exp(jnp.maximum(s, 0.2 * s) - m_shift)

    @pl.loop(0, SLICE // L)
    def _z(i):
        zbuf[pl.ds(i * L, L)] = jnp.zeros((L,), _f32)
    pltpu.sync_copy(zbuf, denom_s.at[pl.ds(nbase, SLICE)])
    plsc.subcore_barrier()

    @pl.loop(0, DEN_CH)
    def _den(j):
        base = (sid * DEN_CH + j) * CH
        pltpu.sync_copy(src_hbm.at[pl.ds(base, CH)], srcc)
        pltpu.sync_copy(dst_hbm.at[pl.ds(base, CH)], dstc)
        for g in range(CH // L):
            _, ex = _edge_ex(g)
            exb[pl.ds(g * L, L)] = ex
        pltpu.sync_copy(exb, denom_s.at[dstc], add=True)
    plsc.subcore_barrier()

    pltpu.sync_copy(denom_s.at[pl.ds(nbase, SLICE)], dbuf)
    pltpu.sync_copy(h_hbm.at[pl.ds(nbase, SLICE)], hbuf)

    @pl.loop(0, SLICE // L)
    def _inv(i):
        s = as_tab[pl.ds(nbase + i * L, L)] + ad_tab[pl.ds(nbase + i * L, L)]
        exl = jnp.exp(jnp.maximum(s, 0.2 * s) - m_shift)
        inv = 1.0 / (dbuf[pl.ds(i * L, L)] + exl)
        zbuf[pl.ds(i * L, L)] = inv
        lbuf[pl.ds(i * L, L)] = inv * exl

    @pl.loop(0, SLICE)
    def _lmsg(j):
        sp = plsc.load_gather(lbuf, [jnp.full((L,), j, _i32)])
        for d in range(DH // L):
            hbuf[j, pl.ds(d * L, L)] = hbuf[j, pl.ds(d * L, L)] * sp

    pltpu.sync_copy(hbuf, acc_s.at[pl.ds(nbase, SLICE)])
    pltpu.sync_copy(zbuf, inv_s.at[pl.ds(nbase, SLICE)])
    plsc.subcore_barrier()
    pltpu.sync_copy(inv_s, inv_tab)

    @pl.loop(0, MSG_CH)
    def _msg(j):
        base = (wid * MSG_CH + j) * CH
        pltpu.sync_copy(src_hbm.at[pl.ds(base, CH)], srcc)
        pltpu.sync_copy(dst_hbm.at[pl.ds(base, CH)], dstc)
        cp = pltpu.async_copy(h_hbm.at[srcc], rows, sem)
        for g in range(CH // L):
            didx, ex = _edge_ex(g)
            iv = plsc.load_gather(inv_tab, [didx])
            alpha[pl.ds(g * L, L)] = ex * iv
        cp.wait()

        @pl.loop(0, CH)
        def _scale(jj):
            sp = plsc.load_gather(alpha, [jnp.full((L,), jj, _i32)])
            for d in range(DH // L):
                rows[jj, pl.ds(d * L, L)] = rows[jj, pl.ds(d * L, L)] * sp

        pltpu.sync_copy(rows, acc_s.at[dstc], add=True)
    plsc.subcore_barrier()

    pltpu.sync_copy(acc_s.at[pl.ds(nbase, SLICE)],
                    acc_out.at[cid].at[pl.ds(nbase, SLICE)])


def _row(v):
    return jnp.reshape(v, (1, DH))


@jax.jit
def kernel(x, edge_index, W1, a_src1, a_dst1, b1, W2, a_src2, a_dst2, b2,
           g1, beta1, g2, beta2):
    src = edge_index[0].astype(_i32)
    dst = edge_index[1].astype(_i32)
    padidx = N + (jnp.arange(EPAD - E, dtype=_i32) % (NPAD - N))
    src_p = jnp.concatenate([src, padidx])
    dst_p = jnp.concatenate([dst, padidx])
    x_p = jnp.pad(x, ((0, NPAD - N), (0, 0)))

    a21 = jnp.stack([a_src1, a_dst1], axis=1)
    a22 = jnp.stack([a_src2, a_dst2], axis=1)

    h1, asad1 = _dense_in(x_p, W1.T, a21)
    acc1 = _gat_sc(src_p, dst_p, asad1[:, 0], asad1[:, 1], h1)
    h2, asad2 = _mid(acc1, _row(b1), _row(g1), _row(beta1), W2.T, a22)
    acc2 = _gat_sc(src_p, dst_p, asad2[:, 0], asad2[:, 1], h2)
    y = _fin(acc2, _row(b2), _row(g2), _row(beta2))
    return y[:N]


# speedup vs baseline: 20.7447x; 20.7447x over previous
import functools

import jax
import jax.numpy as jnp
from jax import lax
from jax.experimental import pallas as pl
from jax.experimental.pallas import tpu as pltpu
from jax.experimental.pallas import tpu_sc as plsc

N = 10000
NPAD = 10240
E = 320000
EPAD = 327680
DIN = 128
DH = 64
NC, NS, L = 2, 16, 16
CH = 128
NCHUNK = EPAD // CH
DEN_CH = NCHUNK // NS
MSG_CH = NCHUNK // (NC * NS)
SLICE = NPAD // NS
RB = 8
BLK = NPAD // RB

_f32 = jnp.float32
_i32 = jnp.int32


def _dense_in_body(x_ref, w_ref, a2_ref, h_ref, asad_ref):
    h = jnp.dot(x_ref[...], w_ref[...], preferred_element_type=_f32)
    h_ref[...] = h
    asad_ref[...] = jnp.dot(h, a2_ref[...], preferred_element_type=_f32)


def _dense_in(x, wT, a2):
    return pl.pallas_call(
        _dense_in_body,
        grid=(RB,),
        in_specs=[
            pl.BlockSpec((BLK, DIN), lambda i: (i, 0)),
            pl.BlockSpec((DIN, DH), lambda i: (0, 0)),
            pl.BlockSpec((DH, 2), lambda i: (0, 0)),
        ],
        out_specs=[
            pl.BlockSpec((BLK, DH), lambda i: (i, 0)),
            pl.BlockSpec((BLK, 2), lambda i: (i, 0)),
        ],
        out_shape=[
            jax.ShapeDtypeStruct((NPAD, DH), _f32),
            jax.ShapeDtypeStruct((NPAD, 2), _f32),
        ],
    )(x, wT, a2)


def _post_ln(acc_ref, b_ref, g_ref, be_ref):
    r = acc_ref[0] + acc_ref[1] + b_ref[...]
    r = jnp.maximum(r, 0.0)
    mu = jnp.mean(r, axis=-1, keepdims=True)
    var = jnp.mean((r - mu) ** 2, axis=-1, keepdims=True)
    return (r - mu) / jnp.sqrt(var + 1e-5) * g_ref[...] + be_ref[...]


def _mid_body(acc_ref, b_ref, g_ref, be_ref, w_ref, a2_ref, h_ref, asad_ref):
    ln = _post_ln(acc_ref, b_ref, g_ref, be_ref)
    h = jnp.dot(ln, w_ref[...], preferred_element_type=_f32)
    h_ref[...] = h
    asad_ref[...] = jnp.dot(h, a2_ref[...], preferred_element_type=_f32)


def _mid(acc, b, g, be, wT, a2):
    return pl.pallas_call(
        _mid_body,
        grid=(RB,),
        in_specs=[
            pl.BlockSpec((NC, BLK, DH), lambda i: (0, i, 0)),
            pl.BlockSpec((1, DH), lambda i: (0, 0)),
            pl.BlockSpec((1, DH), lambda i: (0, 0)),
            pl.BlockSpec((1, DH), lambda i: (0, 0)),
            pl.BlockSpec((DH, DH), lambda i: (0, 0)),
            pl.BlockSpec((DH, 2), lambda i: (0, 0)),
        ],
        out_specs=[
            pl.BlockSpec((BLK, DH), lambda i: (i, 0)),
            pl.BlockSpec((BLK, 2), lambda i: (i, 0)),
        ],
        out_shape=[
            jax.ShapeDtypeStruct((NPAD, DH), _f32),
            jax.ShapeDtypeStruct((NPAD, 2), _f32),
        ],
    )(acc, b, g, be, wT, a2)


def _fin_body(acc_ref, b_ref, g_ref, be_ref, y_ref):
    y_ref[...] = _post_ln(acc_ref, b_ref, g_ref, be_ref)


def _fin(acc, b, g, be):
    return pl.pallas_call(
        _fin_body,
        grid=(RB,),
        in_specs=[
            pl.BlockSpec((NC, BLK, DH), lambda i: (0, i, 0)),
            pl.BlockSpec((1, DH), lambda i: (0, 0)),
            pl.BlockSpec((1, DH), lambda i: (0, 0)),
            pl.BlockSpec((1, DH), lambda i: (0, 0)),
        ],
        out_specs=pl.BlockSpec((BLK, DH), lambda i: (i, 0)),
        out_shape=jax.ShapeDtypeStruct((NPAD, DH), _f32),
    )(acc, b, g, be)


_mesh = plsc.VectorSubcoreMesh(core_axis_name="c", subcore_axis_name="s")


@functools.partial(
    pl.kernel,
    mesh=_mesh,
    compiler_params=pltpu.CompilerParams(
        needs_layout_passes=False, use_tc_tiling_on_sc=False),
    out_type=jax.ShapeDtypeStruct((NC, NPAD, DH), _f32),
    scratch_types=[
        pltpu.VMEM((NPAD,), _f32),
        pltpu.VMEM((NPAD,), _f32),
        pltpu.VMEM((NPAD,), _f32),
        pltpu.VMEM((SLICE, DH), _f32),
        pltpu.VMEM((SLICE,), _f32),
        pltpu.VMEM((SLICE,), _f32),
        pltpu.VMEM((SLICE,), _f32),
        pltpu.VMEM((CH,), _i32),
        pltpu.VMEM((CH,), _i32),
        pltpu.VMEM((CH,), _f32),
        pltpu.VMEM((CH,), _f32),
        pltpu.VMEM((CH, DH), _f32),
        pltpu.VMEM_SHARED((NPAD,), _f32),
        pltpu.VMEM_SHARED((NPAD,), _f32),
        pltpu.VMEM_SHARED((NPAD, DH), _f32),
        pltpu.SemaphoreType.DMA,
    ],
)
def _gat_sc(src_hbm, dst_hbm, as_hbm, ad_hbm, h_hbm, acc_out,
            as_tab, ad_tab, inv_tab, hbuf, dbuf, lbuf, zbuf,
            srcc, dstc, exb, alpha, rows,
            denom_s, inv_s, acc_s, sem):
    cid = lax.axis_index("c")
    sid = lax.axis_index("s")
    wid = cid * NS + sid
    nbase = sid * SLICE

    pltpu.sync_copy(as_hbm, as_tab)
    pltpu.sync_copy(ad_hbm, ad_tab)

    def _mstep(i, carry):
        ma, mb = carry
        ma = jnp.maximum(ma, as_tab[pl.ds(i * L, L)])
        mb = jnp.maximum(mb, ad_tab[pl.ds(i * L, L)])
        return ma, mb
    ninf = jnp.full((L,), -3.0e38, _f32)
    ma, mb = lax.fori_loop(0, NPAD // L, _mstep, (ninf, ninf))

    def _allmax(v):
        iota = lax.iota(_i32, L)
        for k in (1, 2, 4, 8):
            zbuf[pl.ds(0, L)] = v
            v = jnp.maximum(v, plsc.load_gather(zbuf, [iota ^ k]))
        return v

    m_shift = jnp.maximum(_allmax(ma) + _allmax(mb), 0.0)

    def _edge_ex(g):
        sidx = srcc[pl.ds(g * L, L)]
        didx = dstc[pl.ds(g * L, L)]
        s = plsc.load_gather(as_tab, [sidx]) + plsc.load_gather(ad_tab, [didx])
        return didx, jnp.exp(jnp.maximum(s, 0.2 * s) - m_shift)

    @pl.loop(0, SLICE // L)
    def _z(i):
        zbuf[pl.ds(i * L, L)] = jnp.zeros((L,), _f32)
    pltpu.sync_copy(zbuf, denom_s.at[pl.ds(nbase, SLICE)])
    plsc.subcore_barrier()

    @pl.loop(0, DEN_CH)
    def _den(j):
        base = (sid * DEN_CH + j) * CH
        pltpu.sync_copy(src_hbm.at[pl.ds(base, CH)], srcc)
        pltpu.sync_copy(dst_hbm.at[pl.ds(base, CH)], dstc)
        for g in range(CH // L):
            _, ex = _edge_ex(g)
            exb[pl.ds(g * L, L)] = ex
        pltpu.sync_copy(exb, denom_s.at[dstc], add=True)
    plsc.subcore_barrier()

    pltpu.sync_copy(denom_s.at[pl.ds(nbase, SLICE)], dbuf)
    pltpu.sync_copy(h_hbm.at[pl.ds(nbase, SLICE)], hbuf)

    seed_scale = jnp.where(cid == 0, 1.0, 0.0).astype(_f32)

    @pl.loop(0, SLICE // L)
    def _inv(i):
        s = as_tab[pl.ds(nbase + i * L, L)] + ad_tab[pl.ds(nbase + i * L, L)]
        exl = jnp.exp(jnp.maximum(s, 0.2 * s) - m_shift)
        inv = 1.0 / (dbuf[pl.ds(i * L, L)] + exl)
        zbuf[pl.ds(i * L, L)] = inv
        lbuf[pl.ds(i * L, L)] = inv * exl * seed_scale

    @pl.loop(0, SLICE)
    def _lmsg(j):
        sp = plsc.load_gather(lbuf, [jnp.full((L,), j, _i32)])
        for d in range(DH // L):
            hbuf[j, pl.ds(d * L, L)] = hbuf[j, pl.ds(d * L, L)] * sp

    pltpu.sync_copy(hbuf, acc_s.at[pl.ds(nbase, SLICE)])
    pltpu.sync_copy(zbuf, inv_s.at[pl.ds(nbase, SLICE)])
    plsc.subcore_barrier()
    pltpu.sync_copy(inv_s, inv_tab)

    @pl.loop(0, MSG_CH)
    def _msg(j):
        base = (wid * MSG_CH + j) * CH
        pltpu.sync_copy(src_hbm.at[pl.ds(base, CH)], srcc)
        pltpu.sync_copy(dst_hbm.at[pl.ds(base, CH)], dstc)
        cp = pltpu.async_copy(h_hbm.at[srcc], rows, sem)
        for g in range(CH // L):
            didx, ex = _edge_ex(g)
            iv = plsc.load_gather(inv_tab, [didx])
            alpha[pl.ds(g * L, L)] = ex * iv
        cp.wait()

        @pl.loop(0, CH)
        def _scale(jj):
            sp = plsc.load_gather(alpha, [jnp.full((L,), jj, _i32)])
            for d in range(DH // L):
                rows[jj, pl.ds(d * L, L)] = rows[jj, pl.ds(d * L, L)] * sp

        pltpu.sync_copy(rows, acc_s.at[dstc], add=True)
    plsc.subcore_barrier()

    pltpu.sync_copy(acc_s.at[pl.ds(nbase, SLICE)],
                    acc_out.at[cid].at[pl.ds(nbase, SLICE)])


def _row(v):
    return jnp.reshape(v, (1, DH))


@jax.jit
def kernel(x, edge_index, W1, a_src1, a_dst1, b1, W2, a_src2, a_dst2, b2,
           g1, beta1, g2, beta2):
    src = edge_index[0].astype(_i32)
    dst = edge_index[1].astype(_i32)
    padidx = N + (jnp.arange(EPAD - E, dtype=_i32) % (NPAD - N))
    src_p = jnp.concatenate([src, padidx])
    dst_p = jnp.concatenate([dst, padidx])
    x_p = jnp.pad(x, ((0, NPAD - N), (0, 0)))

    a21 = jnp.stack([a_src1, a_dst1], axis=1)
    a22 = jnp.stack([a_src2, a_dst2], axis=1)

    h1, asad1 = _dense_in(x_p, W1.T, a21)
    acc1 = _gat_sc(src_p, dst_p, asad1[:, 0], asad1[:, 1], h1)
    h2, asad2 = _mid(acc1, _row(b1), _row(g1), _row(beta1), W2.T, a22)
    acc2 = _gat_sc(src_p, dst_p, asad2[:, 0], asad2[:, 1], h2)
    y = _fin(acc2, _row(b2), _row(g2), _row(beta2))
    return y[:N]


# speedup vs baseline: 38.5406x; 1.8579x over previous
import functools

import jax
import jax.numpy as jnp
from jax import lax
from jax.experimental import pallas as pl
from jax.experimental.pallas import tpu as pltpu
from jax.experimental.pallas import tpu_sc as plsc

N = 10000
NPAD = 10240
E = 320000
EPAD = 327680
DIN = 128
DH = 64
NC, NS, L = 2, 16, 16
CH = 128
NCHUNK = EPAD // CH
DEN_CH = NCHUNK // NS
MSG_CH = NCHUNK // (NC * NS)
SLICE = NPAD // NS
RB = 8
BLK = NPAD // RB
NSLOT = 4
AGRP = 16
SEED_CH = SLICE // CH

_f32 = jnp.float32
_i32 = jnp.int32


def _dense_in_body(x_ref, w_ref, a2_ref, h_ref, asad_ref):
    h = jnp.dot(x_ref[...], w_ref[...], preferred_element_type=_f32)
    h_ref[...] = h
    asad_ref[...] = jnp.dot(h, a2_ref[...], preferred_element_type=_f32)


def _dense_in(x, wT, a2):
    return pl.pallas_call(
        _dense_in_body,
        grid=(RB,),
        in_specs=[
            pl.BlockSpec((BLK, DIN), lambda i: (i, 0)),
            pl.BlockSpec((DIN, DH), lambda i: (0, 0)),
            pl.BlockSpec((DH, 2), lambda i: (0, 0)),
        ],
        out_specs=[
            pl.BlockSpec((BLK, DH), lambda i: (i, 0)),
            pl.BlockSpec((BLK, 2), lambda i: (i, 0)),
        ],
        out_shape=[
            jax.ShapeDtypeStruct((NPAD, DH), _f32),
            jax.ShapeDtypeStruct((NPAD, 2), _f32),
        ],
    )(x, wT, a2)


def _post_ln(acc_ref, dt_ref, b_ref, g_ref, be_ref):
    r = (acc_ref[0] + acc_ref[1]) / dt_ref[...] + b_ref[...]
    r = jnp.maximum(r, 0.0)
    mu = jnp.mean(r, axis=-1, keepdims=True)
    var = jnp.mean((r - mu) ** 2, axis=-1, keepdims=True)
    return (r - mu) / jnp.sqrt(var + 1e-5) * g_ref[...] + be_ref[...]


def _mid_body(acc_ref, dt_ref, b_ref, g_ref, be_ref, w_ref, a2_ref,
              h_ref, asad_ref):
    ln = _post_ln(acc_ref, dt_ref, b_ref, g_ref, be_ref)
    h = jnp.dot(ln, w_ref[...], preferred_element_type=_f32)
    h_ref[...] = h
    asad_ref[...] = jnp.dot(h, a2_ref[...], preferred_element_type=_f32)


def _mid(acc, dt, b, g, be, wT, a2):
    return pl.pallas_call(
        _mid_body,
        grid=(RB,),
        in_specs=[
            pl.BlockSpec((NC, BLK, DH), lambda i: (0, i, 0)),
            pl.BlockSpec((BLK, 1), lambda i: (i, 0)),
            pl.BlockSpec((1, DH), lambda i: (0, 0)),
            pl.BlockSpec((1, DH), lambda i: (0, 0)),
            pl.BlockSpec((1, DH), lambda i: (0, 0)),
            pl.BlockSpec((DH, DH), lambda i: (0, 0)),
            pl.BlockSpec((DH, 2), lambda i: (0, 0)),
        ],
        out_specs=[
            pl.BlockSpec((BLK, DH), lambda i: (i, 0)),
            pl.BlockSpec((BLK, 2), lambda i: (i, 0)),
        ],
        out_shape=[
            jax.ShapeDtypeStruct((NPAD, DH), _f32),
            jax.ShapeDtypeStruct((NPAD, 2), _f32),
        ],
    )(acc, dt, b, g, be, wT, a2)


def _fin_body(acc_ref, dt_ref, b_ref, g_ref, be_ref, y_ref):
    y_ref[...] = _post_ln(acc_ref, dt_ref, b_ref, g_ref, be_ref)


def _fin(acc, dt, b, g, be):
    return pl.pallas_call(
        _fin_body,
        grid=(RB,),
        in_specs=[
            pl.BlockSpec((NC, BLK, DH), lambda i: (0, i, 0)),
            pl.BlockSpec((BLK, 1), lambda i: (i, 0)),
            pl.BlockSpec((1, DH), lambda i: (0, 0)),
            pl.BlockSpec((1, DH), lambda i: (0, 0)),
            pl.BlockSpec((1, DH), lambda i: (0, 0)),
        ],
        out_specs=pl.BlockSpec((BLK, DH), lambda i: (i, 0)),
        out_shape=jax.ShapeDtypeStruct((NPAD, DH), _f32),
    )(acc, dt, b, g, be)


_mesh = plsc.VectorSubcoreMesh(core_axis_name="c", subcore_axis_name="s")


@functools.partial(
    pl.kernel,
    mesh=_mesh,
    compiler_params=pltpu.CompilerParams(
        needs_layout_passes=False, use_tc_tiling_on_sc=False),
    out_type=[
        jax.ShapeDtypeStruct((NC, NPAD, DH), _f32),
        jax.ShapeDtypeStruct((NPAD,), _f32),
    ],
    scratch_types=[
        pltpu.VMEM((NPAD,), _f32),
        pltpu.VMEM((NPAD,), _f32),
        pltpu.VMEM((MSG_CH, CH), _i32),
        pltpu.VMEM((MSG_CH, CH), _i32),
        pltpu.VMEM((AGRP, CH), _i32),
        pltpu.VMEM((AGRP, CH), _i32),
        pltpu.VMEM((AGRP, CH), _f32),
        pltpu.VMEM((NSLOT, CH), _f32),
        pltpu.VMEM((SLICE,), _f32),
        pltpu.VMEM((SLICE,), _f32),
        pltpu.VMEM((CH, DH), _f32),
        pltpu.VMEM((CH, DH), _f32),
        pltpu.VMEM((CH, DH), _f32),
        pltpu.VMEM((CH, DH), _f32),
        pltpu.VMEM_SHARED((NPAD,), _f32),
        pltpu.VMEM_SHARED((NPAD, DH), _f32),
        pltpu.SemaphoreType.DMA,
        pltpu.SemaphoreType.DMA,
        pltpu.SemaphoreType.DMA,
        pltpu.SemaphoreType.DMA,
        pltpu.SemaphoreType.DMA,
        pltpu.SemaphoreType.DMA,
        pltpu.SemaphoreType.DMA,
        pltpu.SemaphoreType.DMA,
        pltpu.SemaphoreType.DMA,
    ],
)
def _gat_sc(src_hbm, dst_hbm, as_hbm, ad_hbm, h_hbm, acc_out, dtot_out,
            as_tab, ad_tab, s80, d80, s16, d16, e16, a4,
            dbuf, lbuf, rows0, rows1, rows2, rows3,
            denom_s, acc_s,
            asem, gsem0, gsem1, gsem2, gsem3, ssem0, ssem1, ssem2, ssem3):
    cid = lax.axis_index("c")
    sid = lax.axis_index("s")
    wid = cid * NS + sid
    nbase = sid * SLICE
    rows = (rows0, rows1, rows2, rows3)
    gsem = (gsem0, gsem1, gsem2, gsem3)
    ssem = (ssem0, ssem1, ssem2, ssem3)

    pltpu.sync_copy(as_hbm, as_tab)
    pltpu.sync_copy(ad_hbm, ad_tab)

    def _mstep(i, carry):
        ma, mb = carry
        ma = jnp.maximum(ma, as_tab[pl.ds(i * L, L)])
        mb = jnp.maximum(mb, ad_tab[pl.ds(i * L, L)])
        return ma, mb
    ninf = jnp.full((L,), -3.0e38, _f32)
    ma, mb = lax.fori_loop(0, NPAD // L, _mstep, (ninf, ninf))

    def _allmax(v):
        iota = lax.iota(_i32, L)
        for k in (1, 2, 4, 8):
            lbuf[pl.ds(0, L)] = v
            v = jnp.maximum(v, plsc.load_gather(lbuf, [iota ^ k]))
        return v

    m_shift = jnp.maximum(_allmax(ma) + _allmax(mb), 0.0)

    @pl.loop(0, SLICE // L)
    def _z(i):
        lbuf[pl.ds(i * L, L)] = jnp.zeros((L,), _f32)
    pltpu.sync_copy(lbuf, denom_s.at[pl.ds(nbase, SLICE)])
    plsc.subcore_barrier()

    @pl.loop(0, DEN_CH // AGRP)
    def _den(t):
        base = sid * DEN_CH + t * AGRP
        pltpu.sync_copy(src_hbm.at[pl.ds(base, AGRP)], s16)
        pltpu.sync_copy(dst_hbm.at[pl.ds(base, AGRP)], d16)
        for j in range(AGRP):
            for g in range(CH // L):
                sidx = s16[j, pl.ds(g * L, L)]
                didx = d16[j, pl.ds(g * L, L)]
                s = (plsc.load_gather(as_tab, [sidx])
                     + plsc.load_gather(ad_tab, [didx]))
                e16[j, pl.ds(g * L, L)] = jnp.exp(
                    jnp.maximum(s, 0.2 * s) - m_shift)
            pltpu.make_async_copy(e16.at[j], denom_s.at[d16.at[j]],
                                  asem).start(add=True)
        for j in range(AGRP):
            pltpu.make_async_copy(e16.at[j], denom_s.at[d16.at[j]],
                                  asem).wait()
    plsc.subcore_barrier()

    pltpu.sync_copy(denom_s.at[pl.ds(nbase, SLICE)], dbuf)
    seed_scale = jnp.where(cid == 0, 1.0, 0.0).astype(_f32)

    @pl.loop(0, SLICE // L)
    def _inv(i):
        s = as_tab[pl.ds(nbase + i * L, L)] + ad_tab[pl.ds(nbase + i * L, L)]
        exl = jnp.exp(jnp.maximum(s, 0.2 * s) - m_shift)
        dbuf[pl.ds(i * L, L)] = dbuf[pl.ds(i * L, L)] + exl
        lbuf[pl.ds(i * L, L)] = exl * seed_scale

    @pl.when(cid == 0)
    def _():
        pltpu.sync_copy(dbuf, dtot_out.at[pl.ds(nbase, SLICE)])

    for c in range(SEED_CH):
        pltpu.sync_copy(h_hbm.at[pl.ds(nbase + c * CH, CH)], rows0)

        @pl.loop(0, CH)
        def _lmsg(j):
            sp = plsc.load_gather(
                lbuf, [jnp.full((L,), c * CH, _i32) + jnp.full((L,), 1, _i32) * j])
            for d in range(DH // L):
                rows0[j, pl.ds(d * L, L)] = rows0[j, pl.ds(d * L, L)] * sp
        pltpu.sync_copy(rows0, acc_s.at[pl.ds(nbase + c * CH, CH)])
    plsc.subcore_barrier()

    pltpu.sync_copy(src_hbm.at[pl.ds(wid * MSG_CH, MSG_CH)], s80)
    pltpu.sync_copy(dst_hbm.at[pl.ds(wid * MSG_CH, MSG_CH)], d80)

    def _scale_rows(rref, b):
        @pl.loop(0, CH)
        def _s(jj):
            sp = plsc.load_gather(
                a4, [jnp.full((L,), b, _i32), jnp.full((L,), 1, _i32) * jj])
            for d in range(DH // L):
                rref[jj, pl.ds(d * L, L)] = rref[jj, pl.ds(d * L, L)] * sp

    pltpu.make_async_copy(h_hbm.at[s80.at[0]], rows0, gsem0).start()

    @pl.loop(0, MSG_CH // NSLOT)
    def _pb(t):
        for b in range(NSLOT):
            j = t * NSLOT + b
            for g in range(CH // L):
                sidx = s80[j, pl.ds(g * L, L)]
                didx = d80[j, pl.ds(g * L, L)]
                s = (plsc.load_gather(as_tab, [sidx])
                     + plsc.load_gather(ad_tab, [didx]))
                a4[b, pl.ds(g * L, L)] = jnp.exp(
                    jnp.maximum(s, 0.2 * s) - m_shift)
            pltpu.make_async_copy(h_hbm.at[s80.at[j]], rows[b], gsem[b]).wait()
            _scale_rows(rows[b], b)
            pltpu.make_async_copy(rows[b], acc_s.at[d80.at[j]],
                                  ssem[b]).start(add=True)
            b2 = (b + 1) % NSLOT
            jn = j + 1

            @pl.when(jn >= NSLOT)
            def _():
                pltpu.make_async_copy(rows[b2], acc_s.at[d80.at[jn - NSLOT]],
                                      ssem[b2]).wait()

            @pl.when(jn < MSG_CH)
            def _():
                pltpu.make_async_copy(h_hbm.at[s80.at[jn]], rows[b2],
                                      gsem[b2]).start()

    for b in range(1, NSLOT):
        j = MSG_CH - NSLOT + b
        pltpu.make_async_copy(rows[b], acc_s.at[d80.at[j]], ssem[b]).wait()
    plsc.subcore_barrier()

    pltpu.sync_copy(acc_s.at[pl.ds(nbase, SLICE)],
                    acc_out.at[cid].at[pl.ds(nbase, SLICE)])


def _row(v):
    return jnp.reshape(v, (1, DH))


@jax.jit
def kernel(x, edge_index, W1, a_src1, a_dst1, b1, W2, a_src2, a_dst2, b2,
           g1, beta1, g2, beta2):
    src = edge_index[0].astype(_i32)
    dst = edge_index[1].astype(_i32)
    padidx = N + (jnp.arange(EPAD - E, dtype=_i32) % (NPAD - N))
    src_p = jnp.concatenate([src, padidx]).reshape(NCHUNK, CH)
    dst_p = jnp.concatenate([dst, padidx]).reshape(NCHUNK, CH)
    x_p = jnp.pad(x, ((0, NPAD - N), (0, 0)))

    a21 = jnp.stack([a_src1, a_dst1], axis=1)
    a22 = jnp.stack([a_src2, a_dst2], axis=1)

    h1, asad1 = _dense_in(x_p, W1.T, a21)
    acc1, dt1 = _gat_sc(src_p, dst_p, asad1[:, 0], asad1[:, 1], h1)
    h2, asad2 = _mid(acc1, dt1.reshape(NPAD, 1), _row(b1), _row(g1),
                     _row(beta1), W2.T, a22)
    acc2, dt2 = _gat_sc(src_p, dst_p, asad2[:, 0], asad2[:, 1], h2)
    y = _fin(acc2, dt2.reshape(NPAD, 1), _row(b2), _row(g2), _row(beta2))
    return y[:N]


# speedup vs baseline: 44.0929x; 1.1441x over previous
import functools

import jax
import jax.numpy as jnp
from jax import lax
from jax.experimental import pallas as pl
from jax.experimental.pallas import tpu as pltpu
from jax.experimental.pallas import tpu_sc as plsc

N = 10000
NPAD = 10240
E = 320000
EPAD = 327680
DIN = 128
DH = 64
NC, NS, L = 2, 16, 16
CH = 128
NCHUNK = EPAD // CH
DEN_CH = NCHUNK // NS
MSG_CH = NCHUNK // (NC * NS)
SLICE = NPAD // NS
RB = 8
BLK = NPAD // RB
NSLOT = 4
AGRP = 16
SEED_CH = SLICE // CH

_f32 = jnp.float32
_i32 = jnp.int32


def _dense_in_body(x_ref, w_ref, a2_ref, h_ref, asad_ref):
    h = jnp.dot(x_ref[...], w_ref[...], preferred_element_type=_f32)
    h_ref[...] = h
    asad_ref[...] = jnp.dot(h, a2_ref[...], preferred_element_type=_f32)


def _dense_in(x, wT, a2):
    return pl.pallas_call(
        _dense_in_body,
        grid=(RB,),
        in_specs=[
            pl.BlockSpec((BLK, DIN), lambda i: (i, 0)),
            pl.BlockSpec((DIN, DH), lambda i: (0, 0)),
            pl.BlockSpec((DH, 2), lambda i: (0, 0)),
        ],
        out_specs=[
            pl.BlockSpec((BLK, DH), lambda i: (i, 0)),
            pl.BlockSpec((BLK, 2), lambda i: (i, 0)),
        ],
        out_shape=[
            jax.ShapeDtypeStruct((NPAD, DH), _f32),
            jax.ShapeDtypeStruct((NPAD, 2), _f32),
        ],
    )(x, wT, a2)


def _post_ln(acc_ref, dt_ref, asad_ref, m_ref, b_ref, g_ref, be_ref):
    s = asad_ref[:, 0:1] + asad_ref[:, 1:2]
    exl = jnp.exp(jnp.maximum(s, 0.2 * s) - m_ref[0, 0])
    dt = dt_ref[0] + dt_ref[1] + exl
    r = (acc_ref[0] + acc_ref[1]) / dt + b_ref[...]
    r = jnp.maximum(r, 0.0)
    mu = jnp.mean(r, axis=-1, keepdims=True)
    var = jnp.mean((r - mu) ** 2, axis=-1, keepdims=True)
    return (r - mu) / jnp.sqrt(var + 1e-5) * g_ref[...] + be_ref[...]


def _mid_body(acc_ref, dt_ref, asad_ref, m_ref, b_ref, g_ref, be_ref,
              w_ref, a2_ref, h_ref, asadn_ref):
    ln = _post_ln(acc_ref, dt_ref, asad_ref, m_ref, b_ref, g_ref, be_ref)
    h = jnp.dot(ln, w_ref[...], preferred_element_type=_f32)
    h_ref[...] = h
    asadn_ref[...] = jnp.dot(h, a2_ref[...], preferred_element_type=_f32)


_COMBINE_SPECS = [
    pl.BlockSpec((NC, BLK, DH), lambda i: (0, i, 0)),
    pl.BlockSpec((NC, BLK, 1), lambda i: (0, i, 0)),
    pl.BlockSpec((BLK, 2), lambda i: (i, 0)),
    pl.BlockSpec((1, 16), lambda i: (0, 0)),
    pl.BlockSpec((1, DH), lambda i: (0, 0)),
    pl.BlockSpec((1, DH), lambda i: (0, 0)),
    pl.BlockSpec((1, DH), lambda i: (0, 0)),
]


def _mid(acc, dt, asad, m, b, g, be, wT, a2):
    return pl.pallas_call(
        _mid_body,
        grid=(RB,),
        in_specs=_COMBINE_SPECS + [
            pl.BlockSpec((DH, DH), lambda i: (0, 0)),
            pl.BlockSpec((DH, 2), lambda i: (0, 0)),
        ],
        out_specs=[
            pl.BlockSpec((BLK, DH), lambda i: (i, 0)),
            pl.BlockSpec((BLK, 2), lambda i: (i, 0)),
        ],
        out_shape=[
            jax.ShapeDtypeStruct((NPAD, DH), _f32),
            jax.ShapeDtypeStruct((NPAD, 2), _f32),
        ],
    )(acc, dt, asad, m, b, g, be, wT, a2)


def _fin_body(acc_ref, dt_ref, asad_ref, m_ref, b_ref, g_ref, be_ref, y_ref):
    y_ref[...] = _post_ln(acc_ref, dt_ref, asad_ref, m_ref, b_ref, g_ref,
                          be_ref)


def _fin(acc, dt, asad, m, b, g, be):
    return pl.pallas_call(
        _fin_body,
        grid=(RB,),
        in_specs=_COMBINE_SPECS,
        out_specs=pl.BlockSpec((BLK, DH), lambda i: (i, 0)),
        out_shape=jax.ShapeDtypeStruct((NPAD, DH), _f32),
    )(acc, dt, asad, m, b, g, be)


_mesh = plsc.VectorSubcoreMesh(core_axis_name="c", subcore_axis_name="s")


@functools.partial(
    pl.kernel,
    mesh=_mesh,
    compiler_params=pltpu.CompilerParams(
        needs_layout_passes=False, use_tc_tiling_on_sc=False),
    out_type=[
        jax.ShapeDtypeStruct((NC, NPAD, DH), _f32),
        jax.ShapeDtypeStruct((NC, NPAD), _f32),
        jax.ShapeDtypeStruct((16,), _f32),
    ],
    scratch_types=[
        pltpu.VMEM((NPAD,), _f32),
        pltpu.VMEM((NPAD,), _f32),
        pltpu.VMEM((MSG_CH, CH), _i32),
        pltpu.VMEM((MSG_CH, CH), _i32),
        pltpu.VMEM((MSG_CH, CH), _f32),
        pltpu.VMEM((SLICE,), _f32),
        pltpu.VMEM((CH, DH), _f32),
        pltpu.VMEM((CH, DH), _f32),
        pltpu.VMEM((CH, DH), _f32),
        pltpu.VMEM((CH, DH), _f32),
        pltpu.VMEM_SHARED((NPAD,), _f32),
        pltpu.VMEM_SHARED((NPAD, DH), _f32),
        pltpu.SemaphoreType.DMA,
        pltpu.SemaphoreType.DMA,
        pltpu.SemaphoreType.DMA,
        pltpu.SemaphoreType.DMA,
        pltpu.SemaphoreType.DMA,
        pltpu.SemaphoreType.DMA,
        pltpu.SemaphoreType.DMA,
        pltpu.SemaphoreType.DMA,
        pltpu.SemaphoreType.DMA,
    ],
)
def _gat_sc(src_hbm, dst_hbm, as_hbm, ad_hbm, h_hbm,
            acc_out, dtp_out, m_out,
            as_tab, ad_tab, s80, d80, e80, lbuf,
            rows0, rows1, rows2, rows3,
            denom_s, acc_s,
            asem, gsem0, gsem1, gsem2, gsem3, ssem0, ssem1, ssem2, ssem3):
    cid = lax.axis_index("c")
    sid = lax.axis_index("s")
    wid = cid * NS + sid
    nbase = sid * SLICE
    rows = (rows0, rows1, rows2, rows3)
    gsem = (gsem0, gsem1, gsem2, gsem3)
    ssem = (ssem0, ssem1, ssem2, ssem3)

    pltpu.sync_copy(as_hbm, as_tab)
    pltpu.sync_copy(ad_hbm, ad_tab)
    pltpu.sync_copy(src_hbm.at[pl.ds(wid * MSG_CH, MSG_CH)], s80)
    pltpu.sync_copy(dst_hbm.at[pl.ds(wid * MSG_CH, MSG_CH)], d80)

    def _mstep(i, carry):
        ma, mb = carry
        ma = jnp.maximum(ma, as_tab[pl.ds(i * L, L)])
        mb = jnp.maximum(mb, ad_tab[pl.ds(i * L, L)])
        return ma, mb
    ninf = jnp.full((L,), -3.0e38, _f32)
    ma, mb = lax.fori_loop(0, NPAD // L, _mstep, (ninf, ninf))

    def _allmax(v):
        iota = lax.iota(_i32, L)
        for k in (1, 2, 4, 8):
            lbuf[pl.ds(0, L)] = v
            v = jnp.maximum(v, plsc.load_gather(lbuf, [iota ^ k]))
        return v

    m_shift = jnp.maximum(_allmax(ma) + _allmax(mb), 0.0)

    @pl.when(wid == 0)
    def _():
        lbuf[pl.ds(0, L)] = m_shift
        pltpu.sync_copy(lbuf.at[pl.ds(0, L)], m_out)

    @pl.loop(0, SLICE // L)
    def _z(i):
        lbuf[pl.ds(i * L, L)] = jnp.zeros((L,), _f32)
    pltpu.sync_copy(lbuf, denom_s.at[pl.ds(nbase, SLICE)])
    plsc.subcore_barrier()

    @pl.loop(0, MSG_CH // AGRP)
    def _den(t):
        for b in range(AGRP):
            j = t * AGRP + b
            for g in range(CH // L):
                sidx = s80[j, pl.ds(g * L, L)]
                didx = d80[j, pl.ds(g * L, L)]
                s = (plsc.load_gather(as_tab, [sidx])
                     + plsc.load_gather(ad_tab, [didx]))
                e80[j, pl.ds(g * L, L)] = jnp.exp(
                    jnp.maximum(s, 0.2 * s) - m_shift)
            pltpu.make_async_copy(e80.at[j], denom_s.at[d80.at[j]],
                                  asem).start(add=True)
        for b in range(AGRP):
            j = t * AGRP + b
            pltpu.make_async_copy(e80.at[j], denom_s.at[d80.at[j]],
                                  asem).wait()

    seed_scale = jnp.where(cid == 0, 1.0, 0.0).astype(_f32)

    @pl.loop(0, SLICE // L)
    def _exl(i):
        s = as_tab[pl.ds(nbase + i * L, L)] + ad_tab[pl.ds(nbase + i * L, L)]
        exl = jnp.exp(jnp.maximum(s, 0.2 * s) - m_shift)
        lbuf[pl.ds(i * L, L)] = exl * seed_scale

    for c in range(SEED_CH):
        pltpu.sync_copy(h_hbm.at[pl.ds(nbase + c * CH, CH)], rows0)

        @pl.loop(0, CH)
        def _lmsg(j):
            sp = plsc.load_gather(
                lbuf, [jnp.full((L,), c * CH, _i32) + jnp.full((L,), 1, _i32) * j])
            for d in range(DH // L):
                rows0[j, pl.ds(d * L, L)] = rows0[j, pl.ds(d * L, L)] * sp
        pltpu.sync_copy(rows0, acc_s.at[pl.ds(nbase + c * CH, CH)])
    plsc.subcore_barrier()
    pltpu.sync_copy(denom_s.at[pl.ds(nbase, SLICE)],
                    dtp_out.at[cid].at[pl.ds(nbase, SLICE)])

    def _scale_rows(rref, j):
        @pl.loop(0, CH)
        def _s(jj):
            sp = plsc.load_gather(
                e80, [jnp.full((L,), 1, _i32) * j, jnp.full((L,), 1, _i32) * jj])
            for d in range(DH // L):
                rref[jj, pl.ds(d * L, L)] = rref[jj, pl.ds(d * L, L)] * sp

    pltpu.make_async_copy(h_hbm.at[s80.at[0]], rows0, gsem0).start()

    @pl.loop(0, MSG_CH // NSLOT)
    def _pb(t):
        for b in range(NSLOT):
            j = t * NSLOT + b
            pltpu.make_async_copy(h_hbm.at[s80.at[j]], rows[b], gsem[b]).wait()
            _scale_rows(rows[b], j)
            pltpu.make_async_copy(rows[b], acc_s.at[d80.at[j]],
                                  ssem[b]).start(add=True)
            b2 = (b + 1) % NSLOT
            jn = j + 1

            @pl.when(jn >= NSLOT)
            def _():
                pltpu.make_async_copy(rows[b2], acc_s.at[d80.at[jn - NSLOT]],
                                      ssem[b2]).wait()

            @pl.when(jn < MSG_CH)
            def _():
                pltpu.make_async_copy(h_hbm.at[s80.at[jn]], rows[b2],
                                      gsem[b2]).start()

    for b in range(1, NSLOT):
        j = MSG_CH - NSLOT + b
        pltpu.make_async_copy(rows[b], acc_s.at[d80.at[j]], ssem[b]).wait()
    plsc.subcore_barrier()

    pltpu.sync_copy(acc_s.at[pl.ds(nbase, SLICE)],
                    acc_out.at[cid].at[pl.ds(nbase, SLICE)])


def _row(v):
    return jnp.reshape(v, (1, DH))


@jax.jit
def kernel(x, edge_index, W1, a_src1, a_dst1, b1, W2, a_src2, a_dst2, b2,
           g1, beta1, g2, beta2):
    src = edge_index[0].astype(_i32)
    dst = edge_index[1].astype(_i32)
    padidx = N + (jnp.arange(EPAD - E, dtype=_i32) % (NPAD - N))
    src_p = jnp.concatenate([src, padidx]).reshape(NCHUNK, CH)
    dst_p = jnp.concatenate([dst, padidx]).reshape(NCHUNK, CH)
    x_p = jnp.pad(x, ((0, NPAD - N), (0, 0)))

    a21 = jnp.stack([a_src1, a_dst1], axis=1)
    a22 = jnp.stack([a_src2, a_dst2], axis=1)

    h1, asad1 = _dense_in(x_p, W1.T, a21)
    acc1, dtp1, m1 = _gat_sc(src_p, dst_p, asad1[:, 0], asad1[:, 1], h1)
    h2, asad2 = _mid(acc1, dtp1.reshape(NC, NPAD, 1), asad1,
                     m1.reshape(1, 16), _row(b1), _row(g1), _row(beta1),
                     W2.T, a22)
    acc2, dtp2, m2 = _gat_sc(src_p, dst_p, asad2[:, 0], asad2[:, 1], h2)
    y = _fin(acc2, dtp2.reshape(NC, NPAD, 1), asad2, m2.reshape(1, 16),
             _row(b2), _row(g2), _row(beta2))
    return y[:N]
